# baseline (speedup 1.0000x reference)
import functools

import jax
import jax.numpy as jnp
from jax import lax
from jax.experimental import pallas as pl
from jax.experimental.pallas import tpu as pltpu
from jax.experimental.pallas import tpu_sc as plsc

_NUM_ENTITIES = 1000000
_NUM_RELATIONS = 1000
_EMBED_DIM = 64
_BATCH = 16384

_NC = 2
_NS = 16
_L = 16
_NW = _NC * _NS
_BPW = _BATCH // _NW
_HALF = _BPW // 2
_NG = _HALF // _L
_DBLK = _EMBED_DIM // _L

_GDN = lax.GatherDimensionNumbers(
    offset_dims=(), collapsed_slice_dims=(0,), start_index_map=(0,))


def _permute(v, idx):
    return lax.gather(v, idx[:, None], _GDN, slice_sizes=(1,),
                      mode=lax.GatherScatterMode.PROMISE_IN_BOUNDS)


def _hsum_tree(vs, lanes):
    for d in (1, 2, 4, 8):
        perm = lanes ^ d
        mask = (lanes & d) != 0
        nxt = []
        for i in range(0, len(vs), 2):
            a = vs[i] + _permute(vs[i], perm)
            b = vs[i + 1] + _permute(vs[i + 1], perm)
            nxt.append(jnp.where(mask, b, a))
        vs = nxt
    return vs[0]


def _dm_body(e1i, reli, e2i, ent, rel, out,
             idx_e1, idx_r, idx_e2, rows_e1, rows_e2, rel_stage,
             relc, out_v, *sems):
    w = lax.axis_index("s") * _NC + lax.axis_index("c")
    s_id = lax.axis_index("s")
    base = w * _BPW

    pltpu.sync_copy(e1i.at[pl.ds(base, _BPW)], idx_e1)
    pltpu.sync_copy(reli.at[pl.ds(base, _BPW)], idx_r)
    pltpu.sync_copy(e2i.at[pl.ds(base, _BPW)], idx_e2)

    _NRT = _NUM_RELATIONS // 8

    def rel_load(c, carry):
        start = jnp.minimum(c * 8, _NRT - 8)
        pltpu.sync_copy(rel.at[pl.ds(start, 8)], rel_stage)
        for s in range(8):
            li = (start + s) * 4
            for r in range(8):
                ti = li + (r >> 1)
                off = (r & 1) * _EMBED_DIM
                for k in range(_DBLK):
                    relc[ti, pl.ds(off + k * _L, _L)] = (
                        rel_stage[s, r, pl.ds(k * _L, _L)])
        return carry

    lax.fori_loop(0, 16, rel_load, 0)

    def issue(h, g):
        gb = h * _HALF + g * _L
        iv1 = idx_e1[pl.ds(gb, _L)]
        iv2 = idx_e2[pl.ds(gb, _L)]
        for t in range(_L):
            kk = g * (_L // 2) + (t // 2)
            half = pl.ds((t % 2) * _EMBED_DIM, _EMBED_DIM)
            pltpu.async_copy(ent.at[iv1[t]], rows_e1.at[kk, half], sems[t % 8])
            pltpu.async_copy(ent.at[iv2[t]], rows_e2.at[kk, half], sems[8 + t % 8])

    def drain(g):
        for t in range(_L):
            kk = g * (_L // 2) + (t // 2)
            half = pl.ds((t % 2) * _EMBED_DIM, _EMBED_DIM)
            pltpu.make_async_copy(ent.at[0], rows_e1.at[kk, half], sems[t % 8]).wait()
            pltpu.make_async_copy(ent.at[0], rows_e2.at[kk, half], sems[8 + t % 8]).wait()

    lanes = lax.iota(jnp.int32, _L)

    def compute_blk(h, g, carry):
        bb = h * _HALF + g * _L
        ivr = idx_r[pl.ds(bb, _L)]
        rt = ivr >> 1
        ro = (ivr & 1) * _EMBED_DIM
        partials = []
        for i in range(_L):
            r2 = g * (_L // 2) + (i // 2)
            off = (i % 2) * _EMBED_DIM
            ti = rt[i]
            toff = ro[i]
            sl = pl.ds(off, _L)
            acc = rows_e1[r2, sl] * relc[ti, pl.ds(toff, _L)] * rows_e2[r2, sl]
            for k in range(1, _DBLK):
                sl = pl.ds(off + k * _L, _L)
                acc = acc + (rows_e1[r2, sl] * relc[ti, pl.ds(toff + k * _L, _L)]
                             * rows_e2[r2, sl])
            partials.append(acc)
        tot = _hsum_tree(partials, lanes)
        y = 1.0 / (1.0 + jnp.exp(-tot))
        out_v[pl.ds(bb, _L)] = y
        return carry

    issue(0, 0)
    issue(0, 1)

    def fetch0(g, carry):
        issue(0, g)
        drain(g - 2)
        return carry

    lax.fori_loop(2, _NG, fetch0, 0)
    drain(_NG - 2)
    drain(_NG - 1)

    def overlap(g, carry):
        issue(1, g)
        return compute_blk(0, g, carry)

    lax.fori_loop(0, _NG, overlap, 0)

    def tail(g, carry):
        drain(g)
        return compute_blk(1, g, carry)

    lax.fori_loop(0, _NG, tail, 0)

    pltpu.sync_copy(out_v, out.at[pl.ds(base, _BPW)])


@functools.partial(
    pl.kernel,
    out_type=jax.ShapeDtypeStruct((_BATCH,), jnp.float32),
    mesh=plsc.VectorSubcoreMesh(core_axis_name="c", subcore_axis_name="s"),
    compiler_params=pltpu.CompilerParams(use_tc_tiling_on_sc=True),
    scratch_types=[
        pltpu.VMEM((_BPW,), jnp.int32),
        pltpu.VMEM((_BPW,), jnp.int32),
        pltpu.VMEM((_BPW,), jnp.int32),
        pltpu.VMEM((_HALF // 2, 2 * _EMBED_DIM), jnp.float32),
        pltpu.VMEM((_HALF // 2, 2 * _EMBED_DIM), jnp.float32),
        pltpu.VMEM((8, 8, _EMBED_DIM), jnp.float32),
        pltpu.VMEM((_NUM_RELATIONS // 2, 2 * _EMBED_DIM), jnp.float32),
        pltpu.VMEM((_BPW,), jnp.float32),
    ] + [pltpu.SemaphoreType.DMA] * 16,
)
def _dm_sc(e1i, reli, e2i, ent, rel, out, *scratch):
    _dm_body(e1i, reli, e2i, ent, rel, out, *scratch)


def kernel(e1_idx, rel_idx, e2_idx, entity_embedding, rel_embedding):
    e1i = e1_idx.astype(jnp.int32)
    reli = rel_idx.astype(jnp.int32)
    e2i = e2_idx.astype(jnp.int32)
    rel3 = rel_embedding.reshape(_NUM_RELATIONS // 8, 8, _EMBED_DIM)
    out = _dm_sc(e1i, reli, e2i, entity_embedding, rel3)
    return (out, 0.0)


# speedup vs baseline: 1.1760x; 1.1760x over previous
import functools

import jax
import jax.numpy as jnp
from jax import lax
from jax.experimental import pallas as pl
from jax.experimental.pallas import tpu as pltpu
from jax.experimental.pallas import tpu_sc as plsc

_NUM_ENTITIES = 1000000
_NUM_RELATIONS = 1000
_EMBED_DIM = 64
_BATCH = 16384

_NC = 2
_NS = 16
_L = 16
_NW = _NC * _NS
_BPW = _BATCH // _NW
_DBLK = _EMBED_DIM // _L

_GDN = lax.GatherDimensionNumbers(
    offset_dims=(), collapsed_slice_dims=(0,), start_index_map=(0,))


def _permute(v, idx):
    return lax.gather(v, idx[:, None], _GDN, slice_sizes=(1,),
                      mode=lax.GatherScatterMode.PROMISE_IN_BOUNDS)


def _hsum_tree(vs, lanes):
    for d in (1, 2, 4, 8):
        perm = lanes ^ d
        mask = (lanes & d) != 0
        nxt = []
        for i in range(0, len(vs), 2):
            a = vs[i] + _permute(vs[i], perm)
            b = vs[i + 1] + _permute(vs[i + 1], perm)
            nxt.append(jnp.where(mask, b, a))
        vs = nxt
    return vs[0]


def _dm_body(e1i, reli, e2i, ent, rel, out,
             idx_e1, idx_r, idx_e2, rows_e1, rows_r, rows_e2,
             out_v, sem):
    w = lax.axis_index("s") * _NC + lax.axis_index("c")
    base = w * _BPW

    pltpu.sync_copy(e1i.at[pl.ds(base, _BPW)], idx_e1)
    pltpu.sync_copy(reli.at[pl.ds(base, _BPW)], idx_r)
    pltpu.sync_copy(e2i.at[pl.ds(base, _BPW)], idx_e2)

    def issue(g):
        gb = g * _L
        iv1 = idx_e1[pl.ds(gb, _L)]
        ivr = idx_r[pl.ds(gb, _L)]
        iv2 = idx_e2[pl.ds(gb, _L)]
        for t in range(_L):
            kk = g * (_L // 2) + (t // 2)
            half = pl.ds((t % 2) * _EMBED_DIM, _EMBED_DIM)
            pltpu.async_copy(ent.at[iv1[t]], rows_e1.at[kk, half], sem)
            pltpu.async_copy(rel.at[ivr[t]], rows_r.at[kk, half], sem)
            pltpu.async_copy(ent.at[iv2[t]], rows_e2.at[kk, half], sem)

    def drain(g):
        for t in range(_L):
            kk = g * (_L // 2) + (t // 2)
            half = pl.ds((t % 2) * _EMBED_DIM, _EMBED_DIM)
            pltpu.make_async_copy(ent.at[0], rows_e1.at[kk, half], sem).wait()
            pltpu.make_async_copy(rel.at[0], rows_r.at[kk, half], sem).wait()
            pltpu.make_async_copy(ent.at[0], rows_e2.at[kk, half], sem).wait()

    _NG = _BPW // _L
    issue(0)
    issue(1)

    def fetch(g, carry):
        issue(g)
        drain(g - 2)
        return carry

    lax.fori_loop(2, _NG, fetch, 0)
    drain(_NG - 2)
    drain(_NG - 1)

    lanes = lax.iota(jnp.int32, _L)

    def blk_body(b, carry):
        bb = b * _L
        partials = []
        for i in range(_L):
            r2 = b * (_L // 2) + (i // 2)
            off = (i % 2) * _EMBED_DIM
            sl = pl.ds(off, _L)
            acc = rows_e1[r2, sl] * rows_r[r2, sl] * rows_e2[r2, sl]
            for k in range(1, _DBLK):
                sl = pl.ds(off + k * _L, _L)
                acc = acc + rows_e1[r2, sl] * rows_r[r2, sl] * rows_e2[r2, sl]
            partials.append(acc)
        tot = _hsum_tree(partials, lanes)
        y = 1.0 / (1.0 + jnp.exp(-tot))
        out_v[pl.ds(bb, _L)] = y
        return carry

    lax.fori_loop(0, _BPW // _L, blk_body, 0)

    pltpu.sync_copy(out_v, out.at[pl.ds(base, _BPW)])


@functools.partial(
    pl.kernel,
    out_type=jax.ShapeDtypeStruct((_BATCH,), jnp.float32),
    mesh=plsc.VectorSubcoreMesh(core_axis_name="c", subcore_axis_name="s"),
    compiler_params=pltpu.CompilerParams(use_tc_tiling_on_sc=True),
    scratch_types=[
        pltpu.VMEM((_BPW,), jnp.int32),
        pltpu.VMEM((_BPW,), jnp.int32),
        pltpu.VMEM((_BPW,), jnp.int32),
        pltpu.VMEM((_BPW // 2, 2 * _EMBED_DIM), jnp.float32),
        pltpu.VMEM((_BPW // 2, 2 * _EMBED_DIM), jnp.float32),
        pltpu.VMEM((_BPW // 2, 2 * _EMBED_DIM), jnp.float32),
        pltpu.VMEM((_BPW,), jnp.float32),
        pltpu.SemaphoreType.DMA,
    ],
)
def _dm_sc(e1i, reli, e2i, ent, rel, out, *scratch):
    _dm_body(e1i, reli, e2i, ent, rel, out, *scratch)


def kernel(e1_idx, rel_idx, e2_idx, entity_embedding, rel_embedding):
    e1i = e1_idx.astype(jnp.int32)
    reli = rel_idx.astype(jnp.int32)
    e2i = e2_idx.astype(jnp.int32)
    out = _dm_sc(e1i, reli, e2i, entity_embedding, rel_embedding)
    return (out, 0.0)
